# Initial kernel scaffold; baseline (speedup 1.0000x reference)
import jax
import jax.numpy as jnp
from jax.experimental import pallas as pl


def kernel(inputs, emb_table, W, U, b, Wd, bd):
    raise NotImplementedError("write your pallas kernel here")


# speedup vs baseline: 3.8797x; 3.8797x over previous
import functools

import jax
import jax.numpy as jnp
from jax import lax
from jax.experimental import pallas as pl
from jax.experimental.pallas import tpu as pltpu
from jax.experimental.pallas import tpu_sc as plsc

_NC = 2
_NS = 16
_NW = _NC * _NS
_GCH = 128


def _sc_gather(table_hbm, idx_hbm, out_hbm, idx_v, rows_v, sem, *, rows_per_w):
    wid = lax.axis_index("s") * _NC + lax.axis_index("c")
    base = wid * rows_per_w
    n_ch = rows_per_w // _GCH

    def body(j, carry):
        off = base + j * _GCH
        pltpu.sync_copy(idx_hbm.at[pl.ds(off, _GCH)], idx_v)
        pltpu.async_copy(table_hbm.at[idx_v], rows_v, sem).wait()
        pltpu.sync_copy(rows_v, out_hbm.at[pl.ds(off, _GCH)])
        return carry

    lax.fori_loop(0, n_ch, body, 0)


def _embedding_gather(emb_table, idx_flat):
    n_rows = idx_flat.shape[0]
    d = emb_table.shape[1]
    assert n_rows % (_NW * _GCH) == 0
    rows_per_w = n_rows // _NW
    mesh = plsc.VectorSubcoreMesh(core_axis_name="c", subcore_axis_name="s")
    kern = pl.kernel(
        functools.partial(_sc_gather, rows_per_w=rows_per_w),
        out_type=jax.ShapeDtypeStruct((n_rows, d), jnp.float32),
        mesh=mesh,
        scratch_types=[
            pltpu.VMEM((_GCH,), jnp.int32),
            pltpu.VMEM((_GCH, d), jnp.float32),
            pltpu.SemaphoreType.DMA,
        ],
    )
    return kern(emb_table, idx_flat)


def _lstm_body(x_ref, w_ref, u_ref, b_ref, wd_ref, bd_ref, out_ref, h_ref, c_ref):
    t = pl.program_id(0)
    nh = h_ref.shape[1]

    @pl.when(t == 0)
    def _():
        h_ref[...] = jnp.zeros_like(h_ref)
        c_ref[...] = jnp.zeros_like(c_ref)

    x = x_ref[0]
    h = h_ref[...]
    c = c_ref[...]
    z = (
        jnp.dot(x, w_ref[...], preferred_element_type=jnp.float32)
        + jnp.dot(h, u_ref[...], preferred_element_type=jnp.float32)
        + b_ref[...]
    )
    gi = jax.nn.sigmoid(z[:, :nh])
    gf = jax.nn.sigmoid(z[:, nh : 2 * nh])
    gg = jnp.tanh(z[:, 2 * nh : 3 * nh])
    go = jax.nn.sigmoid(z[:, 3 * nh :])
    c_new = gf * c + gi * gg
    h_new = go * jnp.tanh(c_new)
    c_ref[...] = c_new
    h_ref[...] = h_new

    @pl.when(t == pl.num_programs(0) - 1)
    def _():
        logits = (
            jnp.dot(h_new, wd_ref[...], preferred_element_type=jnp.float32)
            + bd_ref[...]
        )
        m = jnp.max(logits, axis=-1, keepdims=True)
        e = jnp.exp(logits - m)
        out_ref[...] = e / jnp.sum(e, axis=-1, keepdims=True)


def _lstm_softmax(xT, W, U, b2, Wdp, bdp):
    T, B, D = xT.shape
    H4 = W.shape[1]
    H = H4 // 4
    OP = Wdp.shape[1]
    return pl.pallas_call(
        _lstm_body,
        grid=(T,),
        in_specs=[
            pl.BlockSpec((1, B, D), lambda t: (t, 0, 0)),
            pl.BlockSpec((D, H4), lambda t: (0, 0)),
            pl.BlockSpec((H, H4), lambda t: (0, 0)),
            pl.BlockSpec((1, H4), lambda t: (0, 0)),
            pl.BlockSpec((H, OP), lambda t: (0, 0)),
            pl.BlockSpec((1, OP), lambda t: (0, 0)),
        ],
        out_specs=pl.BlockSpec((B, OP), lambda t: (0, 0)),
        out_shape=jax.ShapeDtypeStruct((B, OP), jnp.float32),
        scratch_shapes=[
            pltpu.VMEM((B, H), jnp.float32),
            pltpu.VMEM((B, H), jnp.float32),
        ],
        compiler_params=pltpu.CompilerParams(
            dimension_semantics=("arbitrary",),
        ),
    )(xT, W, U, b2, Wdp, bdp)


def kernel(inputs, emb_table, W, U, b, Wd, bd):
    B, T = inputs.shape
    D = emb_table.shape[1]
    O = Wd.shape[1]
    OP = 1024

    idx_flat = inputs.T.reshape(-1)
    x_flat = _embedding_gather(emb_table, idx_flat)
    xT = x_flat.reshape(T, B, D)

    b2 = b.reshape(1, -1)
    Wdp = jnp.pad(Wd, ((0, 0), (0, OP - O)))
    bdp = jnp.concatenate([bd, jnp.full((OP - O,), -1e30, jnp.float32)]).reshape(1, OP)

    probs = _lstm_softmax(xT, W, U, b2, Wdp, bdp)
    return probs[:, :O]
